# Initial kernel scaffold; baseline (speedup 1.0000x reference)
import jax
import jax.numpy as jnp
from jax.experimental import pallas as pl


def kernel(input_words, output_words, noise_words, inp_embed, out_embed):
    raise NotImplementedError("write your pallas kernel here")


# speedup vs baseline: 1.3503x; 1.3503x over previous
import functools

import jax
import jax.numpy as jnp
from jax import lax
from jax.experimental import pallas as pl
from jax.experimental.pallas import tpu as pltpu
from jax.experimental.pallas import tpu_sc as plsc

B = 16384
S = 5
D = 128
C = 128
NC = 2
NS = 16
NW = NC * NS
R_IN = B // NW
R_NZ = B * S // NW
N_IN = R_IN // C
N_NZ = R_NZ // C


def _body(iw, ow, nzw, iemb, oemb, o_in, o_out, o_nz,
          idx_i, idx_o, idx_n, buf0, buf1, sem0, sem1):
    wid = lax.axis_index("s") * NC + lax.axis_index("c")

    pltpu.sync_copy(iw.at[pl.ds(wid * R_IN, R_IN)], idx_i)
    pltpu.sync_copy(ow.at[pl.ds(wid * R_IN, R_IN)], idx_o)
    pltpu.sync_copy(nzw.at[pl.ds(wid * R_NZ, R_NZ)], idx_n)

    tasks = []
    for j in range(N_IN):
        tasks.append((idx_i, j, iemb, o_in, wid * R_IN + j * C))
    for j in range(N_IN):
        tasks.append((idx_o, j, oemb, o_out, wid * R_IN + j * C))
    for j in range(N_NZ):
        tasks.append((idx_n, j, oemb, o_nz, wid * R_NZ + j * C))

    bufs = (buf0, buf1)
    sems = (sem0, sem1)
    copies = [None, None]
    idx0, j0, tab0, _, _ = tasks[0]
    copies[0] = pltpu.async_copy(
        tab0.at[idx0.at[pl.ds(j0 * C, C)]], bufs[0], sems[0])
    for k in range(len(tasks)):
        if k + 1 < len(tasks):
            idxn, jn, tabn, _, _ = tasks[k + 1]
            b = (k + 1) % 2
            copies[b] = pltpu.async_copy(
                tabn.at[idxn.at[pl.ds(jn * C, C)]], bufs[b], sems[b])
        copies[k % 2].wait()
        _, _, _, oref, obase = tasks[k]
        pltpu.sync_copy(bufs[k % 2], oref.at[pl.ds(obase, C)])


@functools.partial(
    pl.kernel,
    out_type=(
        jax.ShapeDtypeStruct((B, D), jnp.float32),
        jax.ShapeDtypeStruct((B, D), jnp.float32),
        jax.ShapeDtypeStruct((B * S, D), jnp.float32),
    ),
    mesh=plsc.VectorSubcoreMesh(core_axis_name="c", subcore_axis_name="s"),
    scratch_types=[
        pltpu.VMEM((R_IN,), jnp.int32),
        pltpu.VMEM((R_IN,), jnp.int32),
        pltpu.VMEM((R_NZ,), jnp.int32),
        pltpu.VMEM((C, D), jnp.float32),
        pltpu.VMEM((C, D), jnp.float32),
        pltpu.SemaphoreType.DMA,
        pltpu.SemaphoreType.DMA,
    ],
)
def _gather_kernel(*refs):
    _body(*refs)


def kernel(input_words, output_words, noise_words, inp_embed, out_embed):
    iw = input_words.astype(jnp.int32)
    ow = output_words.astype(jnp.int32)
    nz = noise_words.astype(jnp.int32).reshape(B * S)
    o_in, o_out, o_nz = _gather_kernel(iw, ow, nz, inp_embed, out_embed)
    return (o_in, o_out, o_nz.reshape(B, S, D))


# speedup vs baseline: 1.3765x; 1.0194x over previous
import functools

import jax
import jax.numpy as jnp
from jax import lax
from jax.experimental import pallas as pl
from jax.experimental.pallas import tpu as pltpu
from jax.experimental.pallas import tpu_sc as plsc

B = 16384
S = 5
D = 128
C = 128
NC = 2
NS = 16
NW = NC * NS
R_IN = B // NW
R_NZ = B * S // NW
N_IN = R_IN // C
N_NZ = R_NZ // C


NBUF = 6
SKEW = 3


def _body(iw, ow, nzw, iemb, oemb, o_in, o_out, o_nz,
          idx_i, idx_o, idx_n, *rest):
    bufs = rest[:NBUF]
    gsems = rest[NBUF:2 * NBUF]
    wsems = rest[2 * NBUF:3 * NBUF]
    wid = lax.axis_index("s") * NC + lax.axis_index("c")

    pltpu.sync_copy(iw.at[pl.ds(wid * R_IN, R_IN)], idx_i)
    pltpu.sync_copy(ow.at[pl.ds(wid * R_IN, R_IN)], idx_o)
    pltpu.sync_copy(nzw.at[pl.ds(wid * R_NZ, R_NZ)], idx_n)

    tasks = []
    for j in range(N_IN):
        tasks.append((idx_i, j, iemb, o_in, wid * R_IN + j * C))
    for j in range(N_IN):
        tasks.append((idx_o, j, oemb, o_out, wid * R_IN + j * C))
    for j in range(N_NZ):
        tasks.append((idx_n, j, oemb, o_nz, wid * R_NZ + j * C))
    nt = len(tasks)

    g = [None] * NBUF
    w = [None] * NBUF

    def fire(k):
        b = k % NBUF
        if w[b] is not None:
            w[b].wait()
            w[b] = None
        idx, j, tab, _, _ = tasks[k]
        g[b] = pltpu.async_copy(
            tab.at[idx.at[pl.ds(j * C, C)]], bufs[b], gsems[b])

    def drain(k):
        b = k % NBUF
        g[b].wait()
        _, _, _, oref, obase = tasks[k]
        w[b] = pltpu.async_copy(bufs[b], oref.at[pl.ds(obase, C)], wsems[b])

    for k in range(nt + SKEW):
        if k < nt:
            fire(k)
        if k >= SKEW:
            drain(k - SKEW)
    for b in range(NBUF):
        if w[b] is not None:
            w[b].wait()


@functools.partial(
    pl.kernel,
    out_type=(
        jax.ShapeDtypeStruct((B, D), jnp.float32),
        jax.ShapeDtypeStruct((B, D), jnp.float32),
        jax.ShapeDtypeStruct((B * S, D), jnp.float32),
    ),
    mesh=plsc.VectorSubcoreMesh(core_axis_name="c", subcore_axis_name="s"),
    scratch_types=[
        pltpu.VMEM((R_IN,), jnp.int32),
        pltpu.VMEM((R_IN,), jnp.int32),
        pltpu.VMEM((R_NZ,), jnp.int32),
        *[pltpu.VMEM((C, D), jnp.float32) for _ in range(NBUF)],
        *[pltpu.SemaphoreType.DMA for _ in range(2 * NBUF)],
    ],
)
def _gather_kernel(*refs):
    _body(*refs)


def kernel(input_words, output_words, noise_words, inp_embed, out_embed):
    iw = input_words.astype(jnp.int32)
    ow = output_words.astype(jnp.int32)
    nz = noise_words.astype(jnp.int32).reshape(B * S)
    o_in, o_out, o_nz = _gather_kernel(iw, ow, nz, inp_embed, out_embed)
    return (o_in, o_out, o_nz.reshape(B, S, D))


# speedup vs baseline: 1.9777x; 1.4367x over previous
import functools

import jax
import jax.numpy as jnp
from jax import lax
from jax.experimental import pallas as pl
from jax.experimental.pallas import tpu as pltpu
from jax.experimental.pallas import tpu_sc as plsc

B = 16384
S = 5
D = 128
C = 128
NC = 2
NS = 16
NW = NC * NS
R_IN = B // NW
R_NZ = B * S // NW
N_IN = R_IN // C
CB = 16
N_NZ = R_IN // CB

NBUF = 4
SKEW = 2


def _body(iw, ow, nzw, iemb, oemb, o_in, o_out, o_nz,
          idx_i, idx_o, idx_n, *rest):
    bufs = rest[:NBUF]
    nbufs = rest[NBUF:2 * NBUF]
    gsems = rest[2 * NBUF:3 * NBUF]
    wsems = rest[3 * NBUF:4 * NBUF]
    wid = lax.axis_index("s") * NC + lax.axis_index("c")

    pltpu.sync_copy(iw.at[pl.ds(wid * R_IN, R_IN)], idx_i)
    pltpu.sync_copy(ow.at[pl.ds(wid * R_IN, R_IN)], idx_o)
    pltpu.sync_copy(nzw.at[pl.ds(wid * R_NZ, R_NZ)], idx_n)

    tasks = []
    for j in range(N_IN):
        tasks.append((idx_i, j * C, C, iemb, o_in, wid * R_IN + j * C, C))
    for j in range(N_IN):
        tasks.append((idx_o, j * C, C, oemb, o_out, wid * R_IN + j * C, C))
    for j in range(N_NZ):
        tasks.append((idx_n, j * CB * S, CB * S, oemb, o_nz,
                      wid * R_IN + j * CB, CB))
    nt = len(tasks)

    g = [None] * NBUF
    w = [None] * NBUF

    def fire(k):
        b = k % NBUF
        if w[b] is not None:
            w[b].wait()
            w[b] = None
        idx, ib, n, tab, _, _, _ = tasks[k]
        g[b] = pltpu.async_copy(
            tab.at[idx.at[pl.ds(ib, n)]],
            bufs[b] if n == C else nbufs[b], gsems[b])

    def drain(k):
        b = k % NBUF
        g[b].wait()
        _, _, n, _, oref, ob, on = tasks[k]
        src = bufs[b] if n == C else nbufs[b].reshape(CB, S, D)
        w[b] = pltpu.async_copy(src, oref.at[pl.ds(ob, on)], wsems[b])

    for k in range(nt + SKEW):
        if k < nt:
            fire(k)
        if k >= SKEW:
            drain(k - SKEW)
    for b in range(NBUF):
        if w[b] is not None:
            w[b].wait()


@functools.partial(
    pl.kernel,
    out_type=(
        jax.ShapeDtypeStruct((B, D), jnp.float32),
        jax.ShapeDtypeStruct((B, D), jnp.float32),
        jax.ShapeDtypeStruct((B, S, D), jnp.float32),
    ),
    mesh=plsc.VectorSubcoreMesh(core_axis_name="c", subcore_axis_name="s"),
    scratch_types=[
        pltpu.VMEM((R_IN,), jnp.int32),
        pltpu.VMEM((R_IN,), jnp.int32),
        pltpu.VMEM((R_NZ,), jnp.int32),
        *[pltpu.VMEM((C, D), jnp.float32) for _ in range(NBUF)],
        *[pltpu.VMEM((CB * S, D), jnp.float32) for _ in range(NBUF)],
        *[pltpu.SemaphoreType.DMA for _ in range(2 * NBUF)],
    ],
)
def _gather_kernel(*refs):
    _body(*refs)


def kernel(input_words, output_words, noise_words, inp_embed, out_embed):
    iw = input_words.astype(jnp.int32)
    ow = output_words.astype(jnp.int32)
    nz = noise_words.astype(jnp.int32).reshape(B * S)
    o_in, o_out, o_nz = _gather_kernel(iw, ow, nz, inp_embed, out_embed)
    return (o_in, o_out, o_nz)


# speedup vs baseline: 3.5411x; 1.7906x over previous
import functools

import jax
import jax.numpy as jnp
from jax import lax
from jax.experimental import pallas as pl
from jax.experimental.pallas import tpu as pltpu
from jax.experimental.pallas import tpu_sc as plsc

B = 16384
S = 5
D = 128
C = 256
NC = 2
NS = 16
NW = NC * NS
R_IN = B // NW
R_NZ = B * S // NW
N_IN = R_IN // C
N_NZ = R_NZ // C

W = 1
NPAIR = 3
PSKEW = 2


def _body(iw, ow, nzw, iemb, oemb, o_in, o_out, o_nz,
          idx_i, idx_o, idx_n, big, *rest):
    gsems = rest[:NPAIR * W]
    wsems = rest[NPAIR * W:NPAIR * W + NPAIR]
    isems = rest[NPAIR * W + NPAIR:NPAIR * W + NPAIR + 3]
    wid = lax.axis_index("s") * NC + lax.axis_index("c")

    ic = pltpu.async_copy(iw.at[pl.ds(wid * R_IN, R_IN)], idx_i, isems[0])
    oc = pltpu.async_copy(ow.at[pl.ds(wid * R_IN, R_IN)], idx_o, isems[1])
    nc = pltpu.async_copy(nzw.at[pl.ds(wid * R_NZ, R_NZ)], idx_n, isems[2])
    idx_ready = {id(idx_i): ic, id(idx_o): oc, id(idx_n): nc}

    pairs = []
    for j in range(0, N_IN, W):
        pairs.append((idx_i, j * C, iemb, o_in, wid * R_IN + j * C))
    for j in range(0, N_IN, W):
        pairs.append((idx_o, j * C, oemb, o_out, wid * R_IN + j * C))
    for j in range(0, N_NZ, W):
        pairs.append((idx_n, j * C, oemb, o_nz, wid * R_NZ + j * C))
    np_ = len(pairs)

    g = [[None] * W for _ in range(NPAIR)]
    w = [None] * NPAIR

    def fire(p):
        r = p % NPAIR
        if w[r] is not None:
            w[r].wait()
            w[r] = None
        idx, ib, tab, _, _ = pairs[p]
        rdy = idx_ready.pop(id(idx), None)
        if rdy is not None:
            rdy.wait()
        for u in range(W):
            g[r][u] = pltpu.async_copy(
                tab.at[idx.at[pl.ds(ib + u * C, C)]],
                big.at[pl.ds((r * W + u) * C, C)], gsems[r * W + u])

    def drain(p):
        r = p % NPAIR
        for u in range(W):
            g[r][u].wait()
        _, _, _, oref, ob = pairs[p]
        w[r] = pltpu.async_copy(
            big.at[pl.ds(r * W * C, W * C)],
            oref.at[pl.ds(ob, W * C)], wsems[r])

    for p in range(np_ + PSKEW):
        if p < np_:
            fire(p)
        if p >= PSKEW:
            drain(p - PSKEW)
    for r in range(NPAIR):
        if w[r] is not None:
            w[r].wait()


@functools.partial(
    pl.kernel,
    out_type=(
        jax.ShapeDtypeStruct((B, D), jnp.float32),
        jax.ShapeDtypeStruct((B, D), jnp.float32),
        jax.ShapeDtypeStruct((S * B, D), jnp.float32),
    ),
    mesh=plsc.VectorSubcoreMesh(core_axis_name="c", subcore_axis_name="s"),
    scratch_types=[
        pltpu.VMEM((R_IN,), jnp.int32),
        pltpu.VMEM((R_IN,), jnp.int32),
        pltpu.VMEM((R_NZ,), jnp.int32),
        pltpu.VMEM((NPAIR * W * C, D), jnp.float32),
        *[pltpu.SemaphoreType.DMA for _ in range(NPAIR * W + NPAIR + 3)],
    ],
)
def _gather_kernel(*refs):
    _body(*refs)


def kernel(input_words, output_words, noise_words, inp_embed, out_embed):
    iw = input_words.astype(jnp.int32)
    ow = output_words.astype(jnp.int32)
    nz = jnp.transpose(noise_words.astype(jnp.int32)).reshape(B * S)
    o_in, o_out, o_nz = _gather_kernel(iw, ow, nz, inp_embed, out_embed)
    return (o_in, o_out, jnp.transpose(o_nz.reshape(S, B, D), (1, 0, 2)))


# speedup vs baseline: 3.5510x; 1.0028x over previous
import functools

import jax
import jax.numpy as jnp
from jax import lax
from jax.experimental import pallas as pl
from jax.experimental.pallas import tpu as pltpu
from jax.experimental.pallas import tpu_sc as plsc

B = 16384
S = 5
D = 128
C = 128
NC = 2
NS = 16
NW = NC * NS
R_IN = B // NW
R_NZ = B * S // NW
N_IN = R_IN // C
N_NZ = R_NZ // C

W = 2
NPAIR = 3
PSKEW = 2


def _body(iw, ow, nzw, iemb, oemb, o_in, o_out, o_nz,
          idx_i, idx_o, idx_n, big, *rest):
    gsems = rest[:NPAIR * W]
    wsems = rest[NPAIR * W:NPAIR * W + NPAIR]
    isems = rest[NPAIR * W + NPAIR:NPAIR * W + NPAIR + 3]
    wid = lax.axis_index("s") * NC + lax.axis_index("c")

    ic = pltpu.async_copy(iw.at[pl.ds(wid * R_IN, R_IN)], idx_i, isems[0])
    oc = pltpu.async_copy(ow.at[pl.ds(wid * R_IN, R_IN)], idx_o, isems[1])
    nc = pltpu.async_copy(nzw.at[pl.ds(wid * R_NZ, R_NZ)], idx_n, isems[2])
    idx_ready = {id(idx_i): ic, id(idx_o): oc, id(idx_n): nc}

    pairs = []
    for j in range(0, N_IN, W):
        pairs.append((idx_i, j * C, iemb, o_in, wid * R_IN + j * C))
    for j in range(0, N_IN, W):
        pairs.append((idx_o, j * C, oemb, o_out, wid * R_IN + j * C))
    for j in range(0, N_NZ, W):
        pairs.append((idx_n, j * C, oemb, o_nz, wid * R_NZ + j * C))
    np_ = len(pairs)

    g = [[None] * W for _ in range(NPAIR)]
    w = [None] * NPAIR

    def fire(p):
        r = p % NPAIR
        if w[r] is not None:
            w[r].wait()
            w[r] = None
        idx, ib, tab, _, _ = pairs[p]
        rdy = idx_ready.pop(id(idx), None)
        if rdy is not None:
            rdy.wait()
        for u in range(W):
            g[r][u] = pltpu.async_copy(
                tab.at[idx.at[pl.ds(ib + u * C, C)]],
                big.at[pl.ds((r * W + u) * C, C)], gsems[r * W + u])

    def drain(p):
        r = p % NPAIR
        for u in range(W):
            g[r][u].wait()
        _, _, _, oref, ob = pairs[p]
        w[r] = pltpu.async_copy(
            big.at[pl.ds(r * W * C, W * C)],
            oref.at[pl.ds(ob, W * C)], wsems[r])

    for p in range(np_ + PSKEW):
        if p < np_:
            fire(p)
        if p >= PSKEW:
            drain(p - PSKEW)
    for r in range(NPAIR):
        if w[r] is not None:
            w[r].wait()


@functools.partial(
    pl.kernel,
    out_type=(
        jax.ShapeDtypeStruct((B, D), jnp.float32),
        jax.ShapeDtypeStruct((B, D), jnp.float32),
        jax.ShapeDtypeStruct((S * B, D), jnp.float32),
    ),
    mesh=plsc.VectorSubcoreMesh(core_axis_name="c", subcore_axis_name="s"),
    scratch_types=[
        pltpu.VMEM((R_IN,), jnp.int32),
        pltpu.VMEM((R_IN,), jnp.int32),
        pltpu.VMEM((R_NZ,), jnp.int32),
        pltpu.VMEM((NPAIR * W * C, D), jnp.float32),
        *[pltpu.SemaphoreType.DMA for _ in range(NPAIR * W + NPAIR + 3)],
    ],
)
def _gather_kernel(*refs):
    _body(*refs)


def kernel(input_words, output_words, noise_words, inp_embed, out_embed):
    iw = input_words.astype(jnp.int32)
    ow = output_words.astype(jnp.int32)
    nz = jnp.transpose(noise_words.astype(jnp.int32)).reshape(B * S)
    o_in, o_out, o_nz = _gather_kernel(iw, ow, nz, inp_embed, out_embed)
    return (o_in, o_out, jnp.transpose(o_nz.reshape(S, B, D), (1, 0, 2)))
